# Initial kernel scaffold; baseline (speedup 1.0000x reference)
import jax
import jax.numpy as jnp
from jax.experimental import pallas as pl


def kernel(mask, nn_ind):
    raise NotImplementedError("write your pallas kernel here")


# speedup vs baseline: 2.7340x; 2.7340x over previous
import functools

import jax
import jax.numpy as jnp
from jax import lax
from jax.experimental import pallas as pl
from jax.experimental.pallas import tpu as pltpu
from jax.experimental.pallas import tpu_sc as plsc

N = 16384
C = 64
K = 8
NC = 2
NS = 16
NW = NC * NS
PPW = N // NW
CHP = 16
PCH = CHP * K
NCHUNK = PPW // CHP


def _sqrt16(x):
    xs = jnp.maximum(x, jnp.float32(1e-12))
    i = plsc.bitcast(xs, jnp.int32)
    i = jnp.int32(0x5F3759DF) - lax.shift_right_logical(i, 1)
    y = plsc.bitcast(i, jnp.float32)
    for _ in range(3):
        y = y * (jnp.float32(1.5) - jnp.float32(0.5) * xs * y * y)
    return x * y


def _sc_body(mask_hbm, idx_hbm, out_hbm, idx_v, rows_v, cen_v, pq_v, out_v, sem):
    cid = lax.axis_index("c")
    sid = lax.axis_index("s")
    wid = sid * NC + cid

    def chunk_body(t, carry):
        pbase = wid * PPW + t * CHP
        jbase = pbase * K
        pltpu.sync_copy(idx_hbm.at[pl.ds(jbase, PCH)], idx_v)
        pltpu.async_copy(mask_hbm.at[idx_v], rows_v, sem).wait()
        pltpu.sync_copy(mask_hbm.at[pl.ds(pbase, CHP)], cen_v)

        lane = lax.iota(jnp.int32, 16)

        def group_body(g, carry2):
            for pp in range(2):
                p = g * 2 + pp
                cvs = [cen_v[p, pl.ds(16 * q, 16)] for q in range(4)]
                for k in range(K):
                    j = p * K + k
                    ss = None
                    for q in range(4):
                        d = cvs[q] - rows_v[j, pl.ds(16 * q, 16)]
                        ss = d * d if ss is None else ss + d * d
                    pq_v[pp * K + k] = ss
            acc = jnp.zeros((16,), jnp.float32)
            for c in range(16):
                col = jnp.full((16,), c, jnp.int32)
                acc = acc + plsc.load_gather(pq_v, [lane, col])
            out_v[pl.ds(g * 16, 16)] = _sqrt16(acc)
            return carry2

        lax.fori_loop(0, PCH // 16, group_body, 0)
        pltpu.sync_copy(out_v, out_hbm.at[pl.ds(jbase, PCH)])
        return carry

    lax.fori_loop(0, NCHUNK, chunk_body, 0)


_sc_kernel = functools.partial(
    pl.kernel,
    mesh=plsc.VectorSubcoreMesh(core_axis_name="c", subcore_axis_name="s"),
    compiler_params=pltpu.CompilerParams(
        needs_layout_passes=False, use_tc_tiling_on_sc=False
    ),
    out_type=jax.ShapeDtypeStruct((N * K,), jnp.float32),
    scratch_types=[
        pltpu.VMEM((PCH,), jnp.int32),
        pltpu.VMEM((PCH, C), jnp.float32),
        pltpu.VMEM((CHP, C), jnp.float32),
        pltpu.VMEM((16, 16), jnp.float32),
        pltpu.VMEM((PCH,), jnp.float32),
        pltpu.SemaphoreType.DMA,
    ],
)(_sc_body)


def _mean_body(x_ref, o_ref):
    s = jnp.sum(x_ref[...], axis=(0, 1), keepdims=True)
    o_ref[...] = s * jnp.float32(1.0 / (N * K))


_mean_kernel = pl.pallas_call(
    _mean_body,
    out_shape=jax.ShapeDtypeStruct((1, 1), jnp.float32),
)


def kernel(mask, nn_ind):
    mask2d = mask[0]
    idx = nn_ind[0].reshape(N * K)
    per_flat = _sc_kernel(mask2d, idx)
    loss = _mean_kernel(per_flat.reshape(N * K // 128, 128))[0, 0]
    return loss, per_flat.reshape(1, N, K)


# speedup vs baseline: 4.1997x; 1.5361x over previous
import functools

import jax
import jax.numpy as jnp
from jax import lax
from jax.experimental import pallas as pl
from jax.experimental.pallas import tpu as pltpu
from jax.experimental.pallas import tpu_sc as plsc

N = 16384
C = 64
K = 8
NC = 2
NS = 16
NW = NC * NS
PPW = N // NW
CHP = 16
PCH = CHP * K
NCHUNK = PPW // CHP
NSUPER = NCHUNK // 2


def _sqrt16(x):
    xs = jnp.maximum(x, jnp.float32(1e-12))
    i = plsc.bitcast(xs, jnp.int32)
    i = jnp.int32(0x5F3759DF) - lax.shift_right_logical(i, 1)
    y = plsc.bitcast(i, jnp.float32)
    for _ in range(2):
        y = y * (jnp.float32(1.5) - jnp.float32(0.5) * xs * y * y)
    return x * y


def _sc_body(mask_hbm, idx_hbm, out_hbm,
             idx_v, cen_v, rows_a, rows_b, pq_a, pq_b, out_v, sem_a, sem_b):
    cid = lax.axis_index("c")
    sid = lax.axis_index("s")
    wid = sid * NC + cid
    pbase = wid * PPW
    lane = lax.iota(jnp.int32, 16)

    pltpu.sync_copy(idx_hbm.at[pl.ds(pbase * K, PPW * K)], idx_v)
    pltpu.sync_copy(mask_hbm.at[pl.ds(pbase, PPW)], cen_v)

    def start_gather(chunk, rows_v, sem):
        src = mask_hbm.at[idx_v.at[pl.ds(chunk * PCH, PCH)]]
        pltpu.async_copy(src, rows_v, sem)

    def wait_gather(rows_v, sem):
        src = mask_hbm.at[idx_v.at[pl.ds(0, PCH)]]
        pltpu.make_async_copy(src, rows_v, sem).wait()

    def compute_chunk(chunk, rows_v):
        cpt = chunk * CHP
        cpr = chunk * PCH

        def group16(g, pq):
            for pp in range(2):
                p = cpt + g * 2 + pp
                jloc = g * 16 + pp * K
                cvs = [cen_v[p, pl.ds(16 * q, 16)] for q in range(4)]
                for k in range(K):
                    d = [cvs[q] - rows_v[jloc + k, pl.ds(16 * q, 16)]
                         for q in range(4)]
                    pq[pp * K + k] = (d[0] * d[0] + d[1] * d[1]) + (
                        d[2] * d[2] + d[3] * d[3])
            acc4 = []
            for b in range(4):
                t = plsc.load_gather(
                    pq, [lane, jnp.full((16,), 4 * b, jnp.int32)])
                for c in range(4 * b + 1, 4 * b + 4):
                    t = t + plsc.load_gather(
                        pq, [lane, jnp.full((16,), c, jnp.int32)])
                acc4.append(t)
            acc = (acc4[0] + acc4[1]) + (acc4[2] + acc4[3])
            out_v[pl.ds(cpr + g * 16, 16)] = _sqrt16(acc)

        def h_body(h, carry):
            group16(2 * h, pq_a)
            group16(2 * h + 1, pq_b)
            return carry

        lax.fori_loop(0, PCH // 32, h_body, 0)

    start_gather(0, rows_a, sem_a)
    start_gather(1, rows_b, sem_b)

    def super_body(s, carry):
        wait_gather(rows_a, sem_a)
        compute_chunk(2 * s, rows_a)

        @pl.when(s != NSUPER - 1)
        def _():
            start_gather(2 * s + 2, rows_a, sem_a)

        wait_gather(rows_b, sem_b)
        compute_chunk(2 * s + 1, rows_b)

        @pl.when(s != NSUPER - 1)
        def _():
            start_gather(2 * s + 3, rows_b, sem_b)

        return carry

    lax.fori_loop(0, NSUPER, super_body, 0)
    pltpu.sync_copy(out_v, out_hbm.at[pl.ds(pbase * K, PPW * K)])


_sc_kernel = functools.partial(
    pl.kernel,
    mesh=plsc.VectorSubcoreMesh(core_axis_name="c", subcore_axis_name="s"),
    compiler_params=pltpu.CompilerParams(
        needs_layout_passes=False, use_tc_tiling_on_sc=False
    ),
    out_type=jax.ShapeDtypeStruct((N * K,), jnp.float32),
    scratch_types=[
        pltpu.VMEM((PPW * K,), jnp.int32),
        pltpu.VMEM((PPW, C), jnp.float32),
        pltpu.VMEM((PCH, C), jnp.float32),
        pltpu.VMEM((PCH, C), jnp.float32),
        pltpu.VMEM((16, 16), jnp.float32),
        pltpu.VMEM((16, 16), jnp.float32),
        pltpu.VMEM((PPW * K,), jnp.float32),
        pltpu.SemaphoreType.DMA,
        pltpu.SemaphoreType.DMA,
    ],
)(_sc_body)


def _mean_body(x_ref, o_ref):
    s = jnp.sum(x_ref[...], axis=(0, 1), keepdims=True)
    o_ref[...] = s * jnp.float32(1.0 / (N * K))


_mean_kernel = pl.pallas_call(
    _mean_body,
    out_shape=jax.ShapeDtypeStruct((1, 1), jnp.float32),
)


def kernel(mask, nn_ind):
    mask2d = mask[0]
    idx = nn_ind[0].reshape(N * K)
    per_flat = _sc_kernel(mask2d, idx)
    loss = _mean_kernel(per_flat.reshape(N * K // 128, 128))[0, 0]
    return loss, per_flat.reshape(1, N, K)
